# baseline (device time: 136786 ns/iter reference)
import functools

import jax
import jax.numpy as jnp
from jax import lax
from jax.experimental import pallas as pl
from jax.experimental.pallas import tpu as pltpu

N_DEV = 32
B, SQ, D = 2, 128, 512
HQ_LOC, HKV_LOC, DH = 8, 2, 64
ROWS = B * SQ
CH = ROWS // N_DEV
N_HOP = N_DEV - 1


def _ring_allreduce(partial):

    def body(p_ref, out_ref, stage_ref, rs_bufs, ag_bufs,
             send_sem, rs_sems, ag_sems):
        my = lax.axis_index("i")
        right = lax.rem(my + 1, N_DEV)
        left = lax.rem(my + N_DEV - 1, N_DEV)

        barrier = pltpu.get_barrier_semaphore()
        for nbr in (left, right):
            pl.semaphore_signal(barrier, inc=1, device_id=(nbr,),
                                device_id_type=pl.DeviceIdType.MESH)
        pl.semaphore_wait(barrier, 2)

        for h in range(N_HOP):
            send_idx = lax.rem(my + (N_DEV - h), N_DEV)
            if h == 0:
                stage_ref[...] = p_ref[pl.ds(send_idx * CH, CH), :]
            else:
                stage_ref[...] = (rs_bufs[h - 1]
                                  + p_ref[pl.ds(send_idx * CH, CH), :])
            rdma = pltpu.make_async_remote_copy(
                src_ref=stage_ref,
                dst_ref=rs_bufs.at[h],
                send_sem=send_sem,
                recv_sem=rs_sems.at[h],
                device_id=(right,),
                device_id_type=pl.DeviceIdType.MESH,
            )
            rdma.start()
            rdma.wait()

        cstar = lax.rem(my + 1, N_DEV)

        for g in range(N_HOP):
            if g == 0:
                stage_ref[...] = (rs_bufs[N_HOP - 1]
                                  + p_ref[pl.ds(cstar * CH, CH), :])
                out_ref[pl.ds(cstar * CH, CH), :] = stage_ref[...]
                src = stage_ref
            else:
                src = ag_bufs.at[g - 1]
            rdma = pltpu.make_async_remote_copy(
                src_ref=src,
                dst_ref=ag_bufs.at[g],
                send_sem=send_sem,
                recv_sem=ag_sems.at[g],
                device_id=(right,),
                device_id_type=pl.DeviceIdType.MESH,
            )
            rdma.start()
            rdma.wait()
            recv_idx = lax.rem(my + (N_DEV - g), N_DEV)
            out_ref[pl.ds(recv_idx * CH, CH), :] = ag_bufs[g]

        @functools.partial(pl.run_scoped, sem2=pltpu.SemaphoreType.REGULAR)
        def _(sem2):
            for nbr in (left, right):
                pl.semaphore_signal(sem2, inc=1, device_id=(nbr,),
                                    device_id_type=pl.DeviceIdType.MESH)
            pl.semaphore_wait(sem2, 2)

    return pl.pallas_call(
        body,
        out_shape=jax.ShapeDtypeStruct((ROWS, D), jnp.float32),
        in_specs=[pl.BlockSpec(memory_space=pltpu.VMEM)],
        out_specs=pl.BlockSpec(memory_space=pltpu.VMEM),
        scratch_shapes=[
            pltpu.VMEM((CH, D), jnp.float32),
            pltpu.VMEM((N_HOP, CH, D), jnp.float32),
            pltpu.VMEM((N_HOP, CH, D), jnp.float32),
            pltpu.SemaphoreType.DMA,
            pltpu.SemaphoreType.DMA((N_HOP,)),
            pltpu.SemaphoreType.DMA((N_HOP,)),
        ],
        compiler_params=pltpu.CompilerParams(collective_id=0),
    )(partial)


def kernel(x, Wq, Wo, K_ext, V_ext):
    my = lax.axis_index("i")
    bf = jnp.bfloat16

    q = jnp.einsum("bsd,df->bsf", x.astype(bf), Wq.astype(bf),
                   preferred_element_type=jnp.float32)
    q = q.reshape(B, SQ, HQ_LOC, DH).astype(bf)

    k_loc = lax.dynamic_slice_in_dim(K_ext, my * HKV_LOC, HKV_LOC, axis=2)
    v_loc = lax.dynamic_slice_in_dim(V_ext, my * HKV_LOC, HKV_LOC, axis=2)
    k_rep = jnp.repeat(k_loc, 4, axis=2).astype(bf)
    v_rep = jnp.repeat(v_loc, 4, axis=2).astype(bf)

    s = jnp.einsum("bihd,bjhd->bhij", q, k_rep,
                   preferred_element_type=jnp.float32) * 0.125
    p = jax.nn.softmax(s, axis=-1)
    o = jnp.einsum("bhij,bjhd->bihd", p.astype(bf), v_rep,
                   preferred_element_type=jnp.float32)
    o = o.reshape(B, SQ, HQ_LOC * DH).astype(bf)

    partial = jnp.einsum("bsf,fd->bsd", o, Wo.astype(bf),
                         preferred_element_type=jnp.float32)

    out = _ring_allreduce(partial.reshape(ROWS, D))
    return out.reshape(B, SQ, D)


# device time: 39179 ns/iter; 3.4913x vs baseline; 3.4913x over previous
import functools

import jax
import jax.numpy as jnp
from jax import lax
from jax.experimental import pallas as pl
from jax.experimental.pallas import tpu as pltpu

N_DEV = 32
B, SQ, D = 2, 128, 512
HQ_LOC, HKV_LOC, DH = 8, 2, 64
ROWS = B * SQ
CH = ROWS // N_DEV
N_HOP = N_DEV - 1


def _butterfly_allreduce(partial):

    def body(p_ref, out_ref, acc_ref, b0, b1, b2, b3, b4,
             send_sem, rs_sems, ag_sems):
        rs_bufs = [b0, b1, b2, b3, b4]
        my = lax.axis_index("i")
        z = my // 8
        p = my % 8
        y = p // 2
        x = (p % 2) ^ (y & 1)

        def ridx(xx, yy, zz):
            return zz * 8 + 2 * yy + (xx ^ (yy & 1))

        partners = [
            ridx(x, y ^ 1, z),
            ridx(x ^ 1, y, z),
            ridx(x, y, z ^ 1),
            ridx(x, y ^ 2, z),
            ridx(x, y, z ^ 2),
        ]
        tbits = [y & 1, x, z & 1, (y >> 1) & 1, (z >> 1) & 1]

        barrier = pltpu.get_barrier_semaphore()
        for pr in partners:
            pl.semaphore_signal(barrier, inc=1, device_id=(pr,),
                                device_id_type=pl.DeviceIdType.MESH)
        pl.semaphore_wait(barrier, 5)

        lo = 0
        for s in range(5):
            half = 128 >> s
            keep_lo = lo + tbits[s] * half
            send_lo = lo + (1 - tbits[s]) * half
            src = p_ref if s == 0 else acc_ref
            rdma = pltpu.make_async_remote_copy(
                src_ref=src.at[pl.ds(send_lo, half)],
                dst_ref=rs_bufs[s],
                send_sem=send_sem,
                recv_sem=rs_sems.at[s],
                device_id=(partners[s],),
                device_id_type=pl.DeviceIdType.MESH,
            )
            rdma.start()
            rdma.wait()
            if s < 4:
                acc_ref[pl.ds(keep_lo, half), :] = (
                    src[pl.ds(keep_lo, half), :] + rs_bufs[s][...])
            else:
                out_ref[pl.ds(keep_lo, half), :] = (
                    acc_ref[pl.ds(keep_lo, half), :] + rs_bufs[s][...])
            lo = keep_lo

        for s in reversed(range(5)):
            glen = 128 >> s
            rdma = pltpu.make_async_remote_copy(
                src_ref=out_ref.at[pl.ds(lo, glen)],
                dst_ref=out_ref.at[pl.ds(lo, glen)],
                send_sem=send_sem,
                recv_sem=ag_sems.at[s],
                device_id=(partners[s],),
                device_id_type=pl.DeviceIdType.MESH,
            )
            rdma.start()
            rdma.wait()
            lo = lo - tbits[s] * glen

    return pl.pallas_call(
        body,
        out_shape=jax.ShapeDtypeStruct((ROWS, D), jnp.float32),
        in_specs=[pl.BlockSpec(memory_space=pltpu.VMEM)],
        out_specs=pl.BlockSpec(memory_space=pltpu.VMEM),
        scratch_shapes=[
            pltpu.VMEM((ROWS, D), jnp.float32),
            pltpu.VMEM((128, D), jnp.float32),
            pltpu.VMEM((64, D), jnp.float32),
            pltpu.VMEM((32, D), jnp.float32),
            pltpu.VMEM((16, D), jnp.float32),
            pltpu.VMEM((8, D), jnp.float32),
            pltpu.SemaphoreType.DMA,
            pltpu.SemaphoreType.DMA((5,)),
            pltpu.SemaphoreType.DMA((5,)),
        ],
        compiler_params=pltpu.CompilerParams(collective_id=0),
    )(partial)


def kernel(x, Wq, Wo, K_ext, V_ext):
    my = lax.axis_index("i")
    bf = jnp.bfloat16

    q = jnp.einsum("bsd,df->bsf", x.astype(bf), Wq.astype(bf),
                   preferred_element_type=jnp.float32)
    q = q.reshape(B, SQ, HQ_LOC, DH).astype(bf)

    k_loc = lax.dynamic_slice_in_dim(K_ext, my * HKV_LOC, HKV_LOC, axis=2)
    v_loc = lax.dynamic_slice_in_dim(V_ext, my * HKV_LOC, HKV_LOC, axis=2)
    k_rep = jnp.repeat(k_loc, 4, axis=2).astype(bf)
    v_rep = jnp.repeat(v_loc, 4, axis=2).astype(bf)

    s = jnp.einsum("bihd,bjhd->bhij", q, k_rep,
                   preferred_element_type=jnp.float32) * 0.125
    p = jax.nn.softmax(s, axis=-1)
    o = jnp.einsum("bhij,bjhd->bihd", p.astype(bf), v_rep,
                   preferred_element_type=jnp.float32)
    o = o.reshape(B, SQ, HQ_LOC * DH).astype(bf)

    partial = jnp.einsum("bsf,fd->bsd", o, Wo.astype(bf),
                         preferred_element_type=jnp.float32)

    out = _butterfly_allreduce(partial.reshape(ROWS, D))
    return out.reshape(B, SQ, D)


# device time: 33712 ns/iter; 4.0575x vs baseline; 1.1622x over previous
import functools

import jax
import jax.numpy as jnp
from jax import lax
from jax.experimental import pallas as pl
from jax.experimental.pallas import tpu as pltpu

N_DEV = 32
B, SQ, D = 2, 128, 512
HQ_LOC, HKV_LOC, DH = 8, 2, 64
ROWS = B * SQ
CH = ROWS // N_DEV
N_HOP = N_DEV - 1


def _butterfly_allreduce(partial):

    def body(p_ref, out_ref, acc_ref, outb_ref,
             b0, b1, b2, b3, b4, s0, s1, s2, s3, s4,
             send_sem, rs_sems, ag_sems):
        rs_bufs = [b0, b1, b2, b3, b4]
        send_bufs = [s0, s1, s2, s3, s4]
        my = lax.axis_index("i")
        z = my // 8
        p = my % 8
        y = p // 2
        x = (p % 2) ^ (y & 1)

        def ridx(xx, yy, zz):
            return zz * 8 + 2 * yy + (xx ^ (yy & 1))

        partners = [
            ridx(x, y ^ 1, z),
            ridx(x ^ 1, y, z),
            ridx(x, y, z ^ 1),
            ridx(x, y ^ 2, z),
            ridx(x, y, z ^ 2),
        ]
        tbits = [y & 1, x, z & 1, (y >> 1) & 1, (z >> 1) & 1]

        barrier = pltpu.get_barrier_semaphore()
        for pr in partners:
            pl.semaphore_signal(barrier, inc=1, device_id=(pr,),
                                device_id_type=pl.DeviceIdType.MESH)
        pl.semaphore_wait(barrier, 5)

        lo = 0
        for s in range(5):
            half = 128 >> s
            keep_lo = lo + tbits[s] * half
            send_lo = lo + (1 - tbits[s]) * half
            src = p_ref if s == 0 else acc_ref
            send_bufs[s][...] = src[pl.ds(send_lo, half), :].astype(jnp.bfloat16)
            rdma = pltpu.make_async_remote_copy(
                src_ref=send_bufs[s],
                dst_ref=rs_bufs[s],
                send_sem=send_sem,
                recv_sem=rs_sems.at[s],
                device_id=(partners[s],),
                device_id_type=pl.DeviceIdType.MESH,
            )
            rdma.start()
            rdma.wait()
            recv = rs_bufs[s][...].astype(jnp.float32)
            if s < 4:
                acc_ref[pl.ds(keep_lo, half), :] = (
                    src[pl.ds(keep_lo, half), :] + recv)
            else:
                outb_ref[pl.ds(keep_lo, half), :] = (
                    acc_ref[pl.ds(keep_lo, half), :] + recv
                ).astype(jnp.bfloat16)
            lo = keep_lo

        for s in reversed(range(5)):
            glen = 128 >> s
            rdma = pltpu.make_async_remote_copy(
                src_ref=outb_ref.at[pl.ds(lo, glen)],
                dst_ref=outb_ref.at[pl.ds(lo, glen)],
                send_sem=send_sem,
                recv_sem=ag_sems.at[s],
                device_id=(partners[s],),
                device_id_type=pl.DeviceIdType.MESH,
            )
            rdma.start()
            rdma.wait()
            lo = lo - tbits[s] * glen

        out_ref[...] = outb_ref[...].astype(jnp.float32)

    return pl.pallas_call(
        body,
        out_shape=jax.ShapeDtypeStruct((ROWS, D), jnp.float32),
        in_specs=[pl.BlockSpec(memory_space=pltpu.VMEM)],
        out_specs=pl.BlockSpec(memory_space=pltpu.VMEM),
        scratch_shapes=[
            pltpu.VMEM((ROWS, D), jnp.float32),
            pltpu.VMEM((ROWS, D), jnp.bfloat16),
            pltpu.VMEM((128, D), jnp.bfloat16),
            pltpu.VMEM((64, D), jnp.bfloat16),
            pltpu.VMEM((32, D), jnp.bfloat16),
            pltpu.VMEM((16, D), jnp.bfloat16),
            pltpu.VMEM((8, D), jnp.bfloat16),
            pltpu.VMEM((128, D), jnp.bfloat16),
            pltpu.VMEM((64, D), jnp.bfloat16),
            pltpu.VMEM((32, D), jnp.bfloat16),
            pltpu.VMEM((16, D), jnp.bfloat16),
            pltpu.VMEM((8, D), jnp.bfloat16),
            pltpu.SemaphoreType.DMA,
            pltpu.SemaphoreType.DMA((5,)),
            pltpu.SemaphoreType.DMA((5,)),
        ],
        compiler_params=pltpu.CompilerParams(collective_id=0),
    )(partial)


def kernel(x, Wq, Wo, K_ext, V_ext):
    my = lax.axis_index("i")
    bf = jnp.bfloat16

    q = jnp.einsum("bsd,df->bsf", x.astype(bf), Wq.astype(bf),
                   preferred_element_type=jnp.float32)
    q = q.reshape(B, SQ, HQ_LOC, DH).astype(bf)

    k_loc = lax.dynamic_slice_in_dim(K_ext, my * HKV_LOC, HKV_LOC, axis=2)
    v_loc = lax.dynamic_slice_in_dim(V_ext, my * HKV_LOC, HKV_LOC, axis=2)
    k_rep = jnp.repeat(k_loc, 4, axis=2).astype(bf)
    v_rep = jnp.repeat(v_loc, 4, axis=2).astype(bf)

    s = jnp.einsum("bihd,bjhd->bhij", q, k_rep,
                   preferred_element_type=jnp.float32) * 0.125
    p = jax.nn.softmax(s, axis=-1)
    o = jnp.einsum("bhij,bjhd->bihd", p.astype(bf), v_rep,
                   preferred_element_type=jnp.float32)
    o = o.reshape(B, SQ, HQ_LOC * DH).astype(bf)

    partial = jnp.einsum("bsf,fd->bsd", o, Wo.astype(bf),
                         preferred_element_type=jnp.float32)

    out = _butterfly_allreduce(partial.reshape(ROWS, D))
    return out.reshape(B, SQ, D)
